# baseline (speedup 1.0000x reference)
import jax
import jax.numpy as jnp
from jax import lax
from jax.experimental import pallas as pl
from jax.experimental.pallas import tpu as pltpu
from jax.experimental.pallas import tpu_sc as plsc

N_NODES = 10000
D = 128
N_EDGES = 320000

NC = 2
NS = 16
NW = NC * NS
CH = 128

EDGES_PER_W = N_EDGES // NW
PHASES = 2
PH_STEPS = 40
SLAB_ROWS = PH_STEPS + 8
IDX_ROWS = PHASES * SLAB_ROWS
ACC_ROWS = 10112
GARBAGE_ROW = N_NODES
ROWS_PER_SUB = ACC_ROWS // NS
OUT_PER_SUB = 624
OUT_TAIL = N_NODES - NS * OUT_PER_SUB


def _sc_scatter_kernel(x_hbm, srcs_hbm, dsts_hbm, zeros_hbm, partials_hbm,
                       src_v, dst_v, acc, b0, b1, g0, g1, s0, s1):
    bufs = (b0, b1)
    gsem = (g0, g1)
    ssem = (s0, s1)
    c = lax.axis_index("c")
    sub = lax.axis_index("s")
    w = c * NS + sub

    pltpu.sync_copy(zeros_hbm.at[pl.ds(sub * ROWS_PER_SUB, ROWS_PER_SUB)],
                    acc.at[pl.ds(sub * ROWS_PER_SUB, ROWS_PER_SUB)])
    plsc.subcore_barrier()

    def fire_gather(j, b):
        pltpu.async_copy(x_hbm.at[src_v.at[j]], bufs[b], gsem[b])

    def wait_gather(j, b):
        pltpu.make_async_copy(x_hbm.at[src_v.at[j]], bufs[b], gsem[b]).wait()

    def fire_scatter(j, b):
        pltpu.async_copy(bufs[b], acc.at[dst_v.at[j]], ssem[b], add=True)

    def wait_scatter(j, b):
        pltpu.make_async_copy(bufs[b], acc.at[dst_v.at[j]], ssem[b]).wait()

    for p in range(PHASES):
        pltpu.sync_copy(srcs_hbm.at[w, pl.ds(p * SLAB_ROWS, SLAB_ROWS)],
                        src_v)
        pltpu.sync_copy(dsts_hbm.at[w, pl.ds(p * SLAB_ROWS, SLAB_ROWS)],
                        dst_v)

        fire_gather(0, 0)
        fire_gather(1, 1)

        @pl.loop(0, PH_STEPS // 2)
        def _group(i):
            for b in range(2):
                j = i * 2 + b
                wait_gather(j, b)
                fire_scatter(j, b)
                wait_scatter(j, b)
                fire_gather(j + 2, b)

        wait_gather(PH_STEPS, 0)
        wait_gather(PH_STEPS + 1, 1)

    plsc.subcore_barrier()

    pltpu.sync_copy(acc.at[pl.ds(sub * OUT_PER_SUB, OUT_PER_SUB)],
                    partials_hbm.at[c, pl.ds(sub * OUT_PER_SUB, OUT_PER_SUB)])

    @pl.when(sub == 0)
    def _tail():
        pltpu.sync_copy(acc.at[pl.ds(NS * OUT_PER_SUB, OUT_TAIL)],
                        partials_hbm.at[c, pl.ds(NS * OUT_PER_SUB, OUT_TAIL)])


def _add_body(a_ref, b_ref, o_ref):
    o_ref[...] = a_ref[...] + b_ref[...]


@jax.jit
def kernel(x, edge_index):
    src = edge_index[0]
    dst = edge_index[1]

    def layout(idx, fill):
        a = jnp.pad(idx.reshape(NW, EDGES_PER_W),
                    ((0, 0), (0, PHASES * PH_STEPS * CH - EDGES_PER_W)),
                    constant_values=fill)
        a = a.reshape(NW, PHASES, PH_STEPS, CH)
        garb = jnp.full((NW, PHASES, SLAB_ROWS - PH_STEPS, CH), fill, jnp.int32)
        return jnp.concatenate([a, garb], axis=2).reshape(NW, IDX_ROWS, CH)

    src2 = layout(src, 0)
    dst2 = layout(dst, GARBAGE_ROW)
    zeros = jnp.zeros((ACC_ROWS, D), jnp.float32)

    mesh = plsc.VectorSubcoreMesh(core_axis_name="c", subcore_axis_name="s")
    partials = pl.kernel(
        _sc_scatter_kernel,
        out_type=jax.ShapeDtypeStruct((NC, N_NODES, D), jnp.float32),
        mesh=mesh,
        scratch_types=[
            pltpu.VMEM((SLAB_ROWS, CH), jnp.int32),
            pltpu.VMEM((SLAB_ROWS, CH), jnp.int32),
            pltpu.VMEM_SHARED((ACC_ROWS, D), jnp.float32),
            pltpu.VMEM((CH, D), jnp.float32),
            pltpu.VMEM((CH, D), jnp.float32),
            pltpu.SemaphoreType.DMA,
            pltpu.SemaphoreType.DMA,
            pltpu.SemaphoreType.DMA,
            pltpu.SemaphoreType.DMA,
        ],
    )(x, src2, dst2, zeros)

    out = pl.pallas_call(
        _add_body,
        grid=(10,),
        in_specs=[pl.BlockSpec((N_NODES // 10, D), lambda i: (i, 0))] * 2,
        out_specs=pl.BlockSpec((N_NODES // 10, D), lambda i: (i, 0)),
        out_shape=jax.ShapeDtypeStruct((N_NODES, D), jnp.float32),
    )(partials[0], partials[1])
    return out


# speedup vs baseline: 1.0001x; 1.0001x over previous
import jax
import jax.numpy as jnp
from jax import lax
from jax.experimental import pallas as pl
from jax.experimental.pallas import tpu as pltpu
from jax.experimental.pallas import tpu_sc as plsc

N_NODES = 10000
D = 128
N_EDGES = 320000

NC = 2
NS = 16
NW = NC * NS
CH = 128

EDGES_PER_W = N_EDGES // NW
PHASES = 2
PH_STEPS = 40
SLAB_ROWS = PH_STEPS + 8
IDX_ROWS = PHASES * SLAB_ROWS
ACC_ROWS = 10112
GARBAGE_ROW = N_NODES
ROWS_PER_SUB = ACC_ROWS // NS
OUT_PER_SUB = 624
OUT_TAIL = N_NODES - NS * OUT_PER_SUB


def _sc_scatter_kernel(x_hbm, srcs_hbm, dsts_hbm, zeros_hbm, partials_hbm,
                       src_v, dst_v, acc, b0, b1, g0, g1, s0, s1):
    bufs = (b0, b1)
    gsem = (g0, g1)
    ssem = (s0, s1)
    c = lax.axis_index("c")
    sub = lax.axis_index("s")
    w = c * NS + sub

    pltpu.sync_copy(zeros_hbm.at[pl.ds(sub * ROWS_PER_SUB, ROWS_PER_SUB)],
                    acc.at[pl.ds(sub * ROWS_PER_SUB, ROWS_PER_SUB)])
    plsc.subcore_barrier()

    def fire_gather(j, b):
        pltpu.async_copy(x_hbm.at[src_v.at[j]], bufs[b], gsem[b])

    def wait_gather(j, b):
        pltpu.make_async_copy(x_hbm.at[src_v.at[j]], bufs[b], gsem[b]).wait()

    def fire_scatter(j, b):
        pltpu.async_copy(bufs[b], acc.at[dst_v.at[j]], ssem[b], add=True)

    def wait_scatter(j, b):
        pltpu.make_async_copy(bufs[b], acc.at[dst_v.at[j]], ssem[b]).wait()

    for p in range(PHASES):
        pltpu.sync_copy(srcs_hbm.at[w, pl.ds(p * SLAB_ROWS, SLAB_ROWS)],
                        src_v)
        pltpu.sync_copy(dsts_hbm.at[w, pl.ds(p * SLAB_ROWS, SLAB_ROWS)],
                        dst_v)

        fire_gather(0, 0)
        fire_gather(1, 1)

        @pl.loop(0, PH_STEPS // 2)
        def _group(i):
            for b in range(2):
                j = i * 2 + b
                wait_gather(j, b)
                pltpu.sync_copy(bufs[b], acc.at[dst_v.at[j]], add=True)
                fire_gather(j + 2, b)

        wait_gather(PH_STEPS, 0)
        wait_gather(PH_STEPS + 1, 1)

    plsc.subcore_barrier()

    pltpu.sync_copy(acc.at[pl.ds(sub * OUT_PER_SUB, OUT_PER_SUB)],
                    partials_hbm.at[c, pl.ds(sub * OUT_PER_SUB, OUT_PER_SUB)])

    @pl.when(sub == 0)
    def _tail():
        pltpu.sync_copy(acc.at[pl.ds(NS * OUT_PER_SUB, OUT_TAIL)],
                        partials_hbm.at[c, pl.ds(NS * OUT_PER_SUB, OUT_TAIL)])


def _add_body(a_ref, b_ref, o_ref):
    o_ref[...] = a_ref[...] + b_ref[...]


@jax.jit
def kernel(x, edge_index):
    src = edge_index[0]
    dst = edge_index[1]

    def layout(idx, fill):
        a = jnp.pad(idx.reshape(NW, EDGES_PER_W),
                    ((0, 0), (0, PHASES * PH_STEPS * CH - EDGES_PER_W)),
                    constant_values=fill)
        a = a.reshape(NW, PHASES, PH_STEPS, CH)
        garb = jnp.full((NW, PHASES, SLAB_ROWS - PH_STEPS, CH), fill, jnp.int32)
        return jnp.concatenate([a, garb], axis=2).reshape(NW, IDX_ROWS, CH)

    src2 = layout(src, 0)
    dst2 = layout(dst, GARBAGE_ROW)
    zeros = jnp.zeros((ACC_ROWS, D), jnp.float32)

    mesh = plsc.VectorSubcoreMesh(core_axis_name="c", subcore_axis_name="s")
    partials = pl.kernel(
        _sc_scatter_kernel,
        out_type=jax.ShapeDtypeStruct((NC, N_NODES, D), jnp.float32),
        mesh=mesh,
        scratch_types=[
            pltpu.VMEM((SLAB_ROWS, CH), jnp.int32),
            pltpu.VMEM((SLAB_ROWS, CH), jnp.int32),
            pltpu.VMEM_SHARED((ACC_ROWS, D), jnp.float32),
            pltpu.VMEM((CH, D), jnp.float32),
            pltpu.VMEM((CH, D), jnp.float32),
            pltpu.SemaphoreType.DMA,
            pltpu.SemaphoreType.DMA,
            pltpu.SemaphoreType.DMA,
            pltpu.SemaphoreType.DMA,
        ],
    )(x, src2, dst2, zeros)

    out = pl.pallas_call(
        _add_body,
        grid=(10,),
        in_specs=[pl.BlockSpec((N_NODES // 10, D), lambda i: (i, 0))] * 2,
        out_specs=pl.BlockSpec((N_NODES // 10, D), lambda i: (i, 0)),
        out_shape=jax.ShapeDtypeStruct((N_NODES, D), jnp.float32),
    )(partials[0], partials[1])
    return out


# speedup vs baseline: 2.0891x; 2.0888x over previous
import jax
import jax.numpy as jnp
from jax import lax
from jax.experimental import pallas as pl
from jax.experimental.pallas import tpu as pltpu
from jax.experimental.pallas import tpu_sc as plsc

N_NODES = 10000
D = 128
N_EDGES = 320000

NC = 2
NS = 16
NW = NC * NS
CH = 128

EDGES_PER_W = N_EDGES // NW
STEPS = 80
IDX_ROWS = STEPS
ACC_ROWS = 10112
GARBAGE_ROW = N_NODES
ROWS_PER_SUB = ACC_ROWS // NS
OUT_PER_SUB = 624
OUT_TAIL = N_NODES - NS * OUT_PER_SUB


def _sc_scatter_kernel(x_hbm, srcs_hbm, dsts_hbm, zeros_hbm, partials_hbm,
                       src_v, dst_v, acc, rows_v, sem):
    c = lax.axis_index("c")
    sub = lax.axis_index("s")
    w = c * NS + sub

    pltpu.sync_copy(zeros_hbm.at[pl.ds(sub * ROWS_PER_SUB, ROWS_PER_SUB)],
                    acc.at[pl.ds(sub * ROWS_PER_SUB, ROWS_PER_SUB)])

    pltpu.sync_copy(srcs_hbm.at[w], src_v)
    pltpu.sync_copy(dsts_hbm.at[w], dst_v)
    plsc.subcore_barrier()

    @pl.loop(0, STEPS // 2)
    def _group(i):
        for b in range(2):
            j = i * 2 + b
            pltpu.async_copy(x_hbm.at[src_v.at[j]], rows_v, sem).wait()
            pltpu.sync_copy(rows_v, acc.at[dst_v.at[j]], add=True)

    plsc.subcore_barrier()

    pltpu.sync_copy(acc.at[pl.ds(sub * OUT_PER_SUB, OUT_PER_SUB)],
                    partials_hbm.at[c, pl.ds(sub * OUT_PER_SUB, OUT_PER_SUB)])

    @pl.when(sub == 0)
    def _tail():
        pltpu.sync_copy(acc.at[pl.ds(NS * OUT_PER_SUB, OUT_TAIL)],
                        partials_hbm.at[c, pl.ds(NS * OUT_PER_SUB, OUT_TAIL)])


def _add_body(a_ref, b_ref, o_ref):
    o_ref[...] = a_ref[...] + b_ref[...]


@jax.jit
def kernel(x, edge_index):
    src = edge_index[0]
    dst = edge_index[1]

    def layout(idx, fill):
        return jnp.pad(idx.reshape(NW, EDGES_PER_W),
                       ((0, 0), (0, IDX_ROWS * CH - EDGES_PER_W)),
                       constant_values=fill).reshape(NW, IDX_ROWS, CH)

    src2 = layout(src, 0)
    dst2 = layout(dst, GARBAGE_ROW)
    zeros = jnp.zeros((ACC_ROWS, D), jnp.float32)

    mesh = plsc.VectorSubcoreMesh(core_axis_name="c", subcore_axis_name="s")
    partials = pl.kernel(
        _sc_scatter_kernel,
        out_type=jax.ShapeDtypeStruct((NC, N_NODES, D), jnp.float32),
        mesh=mesh,
        scratch_types=[
            pltpu.VMEM((IDX_ROWS, CH), jnp.int32),
            pltpu.VMEM((IDX_ROWS, CH), jnp.int32),
            pltpu.VMEM_SHARED((ACC_ROWS, D), jnp.float32),
            pltpu.VMEM((CH, D), jnp.float32),
            pltpu.SemaphoreType.DMA,
        ],
    )(x, src2, dst2, zeros)

    out = pl.pallas_call(
        _add_body,
        grid=(10,),
        in_specs=[pl.BlockSpec((N_NODES // 10, D), lambda i: (i, 0))] * 2,
        out_specs=pl.BlockSpec((N_NODES // 10, D), lambda i: (i, 0)),
        out_shape=jax.ShapeDtypeStruct((N_NODES, D), jnp.float32),
    )(partials[0], partials[1])
    return out
